# initial kernel scaffold (unmeasured)
import jax
import jax.numpy as jnp
from jax import lax
from jax.experimental import pallas as pl
from jax.experimental.pallas import tpu as pltpu

N_DEV = 4
B, Sq, Hq, Dh = 4, 256, 8, 128
D = Hq * Dh
SCALE = 0.08838834764831843

BF = jnp.bfloat16
F32 = jnp.float32


def kernel(x, Wq, Wo, K_ext, V_ext):
    x2 = x.reshape(B * Sq, D)

    def body(x_ref, wq_ref, wo_ref, k_ref, v_ref, out_ref,
             qs, wqs, wos, o_acc, m_acc, l_acc, att_s,
             co, cm, cl, cag,
             so_sem, ro_sem, sm_sem, rm_sem, sl_sem, rl_sem,
             sag_sem, rag_sem):
        my = lax.axis_index("i")
        left = lax.rem(my + 3, N_DEV)
        right = lax.rem(my + 1, N_DEV)

        barrier = pltpu.get_barrier_semaphore()
        for nbr in (left, right):
            pl.semaphore_signal(barrier, inc=1, device_id=(nbr,),
                                device_id_type=pl.DeviceIdType.MESH)
        pl.semaphore_wait(barrier, 2)

        wqs[...] = wq_ref[...].astype(BF)
        wos[...] = wo_ref[...].astype(BF)
        qs[...] = lax.dot_general(
            x_ref[...].astype(BF), wqs[...],
            (((1,), (0,)), ((), ())), preferred_element_type=F32,
        ).astype(BF)

        for b in range(B):
            for h in range(Hq):
                q = qs[b * Sq:(b + 1) * Sq, h * Dh:(h + 1) * Dh]
                k = k_ref[b, :, h, :].astype(BF)
                s = lax.dot_general(
                    q, k, (((1,), (1,)), ((), ())),
                    preferred_element_type=F32) * SCALE
                m = jnp.max(s, axis=1, keepdims=True)
                p = jnp.exp(s - m)
                l = jnp.sum(p, axis=1, keepdims=True)
                v = v_ref[b, :, h, :].astype(BF)
                o = lax.dot_general(
                    p.astype(BF), v, (((1,), (0,)), ((), ())),
                    preferred_element_type=F32)
                o_acc[b * Sq:(b + 1) * Sq, h * Dh:(h + 1) * Dh] = o
                m_acc[b * Sq:(b + 1) * Sq, h:h + 1] = m
                l_acc[b * Sq:(b + 1) * Sq, h:h + 1] = l

        c0 = left
        co[0] = o_acc[pl.ds(c0 * Sq, Sq), :]
        cm[0] = m_acc[pl.ds(c0 * Sq, Sq), :]
        cl[0] = l_acc[pl.ds(c0 * Sq, Sq), :]

        fin = None
        for s in range(N_DEV - 1):
            ss, rs = s % 2, (s + 1) % 2
            rdmas = [
                pltpu.make_async_remote_copy(
                    src_ref=buf.at[ss], dst_ref=buf.at[rs],
                    send_sem=snd.at[ss], recv_sem=rcv.at[rs],
                    device_id=(right,), device_id_type=pl.DeviceIdType.MESH)
                for buf, snd, rcv in (
                    (co, so_sem, ro_sem),
                    (cm, sm_sem, rm_sem),
                    (cl, sl_sem, rl_sem),
                )
            ]
            for r in rdmas:
                r.start()
            for r in rdmas:
                r.wait()

            c_recv = lax.rem(my + (2 - s + N_DEV), N_DEV)
            m_loc = m_acc[pl.ds(c_recv * Sq, Sq), :]
            l_loc = l_acc[pl.ds(c_recv * Sq, Sq), :]
            m_r = cm[rs]
            l_r = cl[rs]
            m_new = jnp.maximum(m_loc, m_r)
            a_loc = jnp.exp(m_loc - m_new)
            a_r = jnp.exp(m_r - m_new)
            l_new = l_loc * a_loc + l_r * a_r
            if s < N_DEV - 2:
                cm[rs] = m_new
                cl[rs] = l_new
            for h in range(Hq):
                cols = pl.ds(h * Dh, Dh)
                o_h = (o_acc[pl.ds(c_recv * Sq, Sq), cols]
                       * a_loc[:, h:h + 1]
                       + co[rs, :, cols] * a_r[:, h:h + 1])
                if s < N_DEV - 2:
                    co[rs, :, cols] = o_h
                else:
                    att_s[:, cols] = (o_h / l_new[:, h:h + 1]).astype(BF)
            if s == N_DEV - 2:
                fin = lax.dot_general(
                    att_s[...], wos[...], (((1,), (0,)), ((), ())),
                    preferred_element_type=F32)

        out_ref[pl.ds(my * Sq, Sq), :] = fin
        cag[0] = fin
        for h in range(N_DEV - 1):
            ss, rs = h % 2, (h + 1) % 2
            rdma = pltpu.make_async_remote_copy(
                src_ref=cag.at[ss], dst_ref=cag.at[rs],
                send_sem=sag_sem.at[ss], recv_sem=rag_sem.at[rs],
                device_id=(right,), device_id_type=pl.DeviceIdType.MESH)
            rdma.start()
            rdma.wait()
            origin = lax.rem(my + (N_DEV - 1 - h), N_DEV)
            out_ref[pl.ds(origin * Sq, Sq), :] = cag[rs]

    out = pl.pallas_call(
        body,
        out_shape=jax.ShapeDtypeStruct((B * Sq, D), F32),
        in_specs=[pl.BlockSpec(memory_space=pltpu.VMEM)] * 5,
        out_specs=pl.BlockSpec(memory_space=pltpu.VMEM),
        scratch_shapes=[
            pltpu.VMEM((B * Sq, D), BF),
            pltpu.VMEM((D, D), BF),
            pltpu.VMEM((D, D), BF),
            pltpu.VMEM((B * Sq, D), F32),
            pltpu.VMEM((B * Sq, Hq), F32),
            pltpu.VMEM((B * Sq, Hq), F32),
            pltpu.VMEM((Sq, D), BF),
            pltpu.VMEM((2, Sq, D), F32),
            pltpu.VMEM((2, Sq, Hq), F32),
            pltpu.VMEM((2, Sq, Hq), F32),
            pltpu.VMEM((2, Sq, D), F32),
            pltpu.SemaphoreType.DMA((2,)),
            pltpu.SemaphoreType.DMA((2,)),
            pltpu.SemaphoreType.DMA((2,)),
            pltpu.SemaphoreType.DMA((2,)),
            pltpu.SemaphoreType.DMA((2,)),
            pltpu.SemaphoreType.DMA((2,)),
            pltpu.SemaphoreType.DMA((2,)),
            pltpu.SemaphoreType.DMA((2,)),
        ],
        compiler_params=pltpu.CompilerParams(
            collective_id=0,
            vmem_limit_bytes=120 * 1024 * 1024,
        ),
    )(x2, Wq, Wo, K_ext, V_ext)
    return out.reshape(B, Sq, D)


# baseline (device time: 148594 ns/iter reference)
import jax
import jax.numpy as jnp
from jax import lax
from jax.experimental import pallas as pl
from jax.experimental.pallas import tpu as pltpu

N_DEV = 4
B, Sq, Hq, Dh = 4, 256, 8, 128
D = Hq * Dh
Skv = 1024
SCALE = 0.08838834764831843

BF = jnp.bfloat16
F32 = jnp.float32


def kernel(x, Wq, Wo, K_ext, V_ext):
    x2 = x.reshape(B * Sq, D)

    def body(x_ref, wq_ref, wo_ref, k_ref, v_ref, out_ref,
             qs, wqs, wos, o_acc, m_acc, l_acc, att_s,
             k_buf, v_buf,
             co, cm, cl, cag,
             kv_sems,
             so_sem, ro_sem, sm_sem, rm_sem, sl_sem, rl_sem,
             sag_sem, rag_sem):
        my = lax.axis_index("i")
        left = lax.rem(my + 3, N_DEV)
        right = lax.rem(my + 1, N_DEV)

        barrier = pltpu.get_barrier_semaphore()
        for nbr in (left, right):
            pl.semaphore_signal(barrier, inc=1, device_id=(nbr,),
                                device_id_type=pl.DeviceIdType.MESH)
        pl.semaphore_wait(barrier, 2)

        wqs[...] = wq_ref[...].astype(BF)
        wos[...] = wo_ref[...].astype(BF)
        for b in range(B):
            rows = pl.ds(b * Sq, Sq)
            qs[rows, :] = lax.dot_general(
                x_ref[rows, :].astype(BF), wqs[...],
                (((1,), (0,)), ((), ())), preferred_element_type=F32,
            ).astype(BF)

        def start_kv(it, slot):
            b, h = divmod(it, Hq)
            pltpu.make_async_copy(
                k_ref.at[b, :, h, :], k_buf.at[slot], kv_sems.at[slot, 0]
            ).start()
            pltpu.make_async_copy(
                v_ref.at[b, :, h, :], v_buf.at[slot], kv_sems.at[slot, 1]
            ).start()

        def wait_kv(slot):
            pltpu.make_async_copy(
                k_ref.at[0, :, 0, :], k_buf.at[slot], kv_sems.at[slot, 0]
            ).wait()
            pltpu.make_async_copy(
                v_ref.at[0, :, 0, :], v_buf.at[slot], kv_sems.at[slot, 1]
            ).wait()

        start_kv(0, 0)
        for it in range(B * Hq):
            b, h = divmod(it, Hq)
            slot = it % 2
            wait_kv(slot)
            if it + 1 < B * Hq:
                start_kv(it + 1, (it + 1) % 2)
            q = qs[b * Sq:(b + 1) * Sq, h * Dh:(h + 1) * Dh]
            k = k_buf[slot].astype(BF)
            s = lax.dot_general(
                q, k, (((1,), (1,)), ((), ())),
                preferred_element_type=F32) * SCALE
            m = jnp.max(s, axis=1, keepdims=True)
            p = jnp.exp(s - m)
            l = jnp.sum(p, axis=1, keepdims=True)
            v = v_buf[slot].astype(BF)
            o = lax.dot_general(
                p.astype(BF), v, (((1,), (0,)), ((), ())),
                preferred_element_type=F32)
            o_acc[b * Sq:(b + 1) * Sq, h * Dh:(h + 1) * Dh] = o
            m_acc[b * Sq:(b + 1) * Sq, h:h + 1] = m
            l_acc[b * Sq:(b + 1) * Sq, h:h + 1] = l

        c0 = left
        co[0] = o_acc[pl.ds(c0 * Sq, Sq), :]
        cm[0] = m_acc[pl.ds(c0 * Sq, Sq), :]
        cl[0] = l_acc[pl.ds(c0 * Sq, Sq), :]

        fin = None
        for s in range(N_DEV - 1):
            ss, rs = s % 2, (s + 1) % 2
            rdmas = [
                pltpu.make_async_remote_copy(
                    src_ref=buf.at[ss], dst_ref=buf.at[rs],
                    send_sem=snd.at[ss], recv_sem=rcv.at[rs],
                    device_id=(right,), device_id_type=pl.DeviceIdType.MESH)
                for buf, snd, rcv in (
                    (co, so_sem, ro_sem),
                    (cm, sm_sem, rm_sem),
                    (cl, sl_sem, rl_sem),
                )
            ]
            for r in rdmas:
                r.start()
            for r in rdmas:
                r.wait()

            c_recv = lax.rem(my + (2 - s + N_DEV), N_DEV)
            m_loc = m_acc[pl.ds(c_recv * Sq, Sq), :]
            l_loc = l_acc[pl.ds(c_recv * Sq, Sq), :]
            m_r = cm[rs]
            l_r = cl[rs]
            m_new = jnp.maximum(m_loc, m_r)
            a_loc = jnp.exp(m_loc - m_new)
            a_r = jnp.exp(m_r - m_new)
            l_new = l_loc * a_loc + l_r * a_r
            if s < N_DEV - 2:
                cm[rs] = m_new
                cl[rs] = l_new
            for h in range(Hq):
                cols = pl.ds(h * Dh, Dh)
                o_h = (o_acc[pl.ds(c_recv * Sq, Sq), cols]
                       * a_loc[:, h:h + 1]
                       + co[rs, :, cols] * a_r[:, h:h + 1])
                if s < N_DEV - 2:
                    co[rs, :, cols] = o_h
                else:
                    att_s[:, cols] = (o_h / l_new[:, h:h + 1]).astype(BF)
            if s == N_DEV - 2:
                fin = lax.dot_general(
                    att_s[...], wos[...], (((1,), (0,)), ((), ())),
                    preferred_element_type=F32)

        out_ref[pl.ds(my * Sq, Sq), :] = fin
        cag[0] = fin
        for h in range(N_DEV - 1):
            ss, rs = h % 2, (h + 1) % 2
            rdma = pltpu.make_async_remote_copy(
                src_ref=cag.at[ss], dst_ref=cag.at[rs],
                send_sem=sag_sem.at[ss], recv_sem=rag_sem.at[rs],
                device_id=(right,), device_id_type=pl.DeviceIdType.MESH)
            rdma.start()
            rdma.wait()
            origin = lax.rem(my + (N_DEV - 1 - h), N_DEV)
            out_ref[pl.ds(origin * Sq, Sq), :] = cag[rs]

    out = pl.pallas_call(
        body,
        out_shape=jax.ShapeDtypeStruct((B * Sq, D), F32),
        in_specs=[
            pl.BlockSpec(memory_space=pltpu.VMEM),
            pl.BlockSpec(memory_space=pltpu.VMEM),
            pl.BlockSpec(memory_space=pltpu.VMEM),
            pl.BlockSpec(memory_space=pltpu.MemorySpace.HBM),
            pl.BlockSpec(memory_space=pltpu.MemorySpace.HBM),
        ],
        out_specs=pl.BlockSpec(memory_space=pltpu.VMEM),
        scratch_shapes=[
            pltpu.VMEM((B * Sq, D), BF),
            pltpu.VMEM((D, D), BF),
            pltpu.VMEM((D, D), BF),
            pltpu.VMEM((B * Sq, D), F32),
            pltpu.VMEM((B * Sq, Hq), F32),
            pltpu.VMEM((B * Sq, Hq), F32),
            pltpu.VMEM((Sq, D), BF),
            pltpu.VMEM((2, Skv, Dh), F32),
            pltpu.VMEM((2, Skv, Dh), F32),
            pltpu.VMEM((2, Sq, D), F32),
            pltpu.VMEM((2, Sq, Hq), F32),
            pltpu.VMEM((2, Sq, Hq), F32),
            pltpu.VMEM((2, Sq, D), F32),
            pltpu.SemaphoreType.DMA((2, 2)),
            pltpu.SemaphoreType.DMA((2,)),
            pltpu.SemaphoreType.DMA((2,)),
            pltpu.SemaphoreType.DMA((2,)),
            pltpu.SemaphoreType.DMA((2,)),
            pltpu.SemaphoreType.DMA((2,)),
            pltpu.SemaphoreType.DMA((2,)),
            pltpu.SemaphoreType.DMA((2,)),
            pltpu.SemaphoreType.DMA((2,)),
        ],
        compiler_params=pltpu.CompilerParams(
            collective_id=0,
            vmem_limit_bytes=60 * 1024 * 1024,
        ),
    )(x2, Wq, Wo, K_ext, V_ext)
    return out.reshape(B, Sq, D)


# device time: 76729 ns/iter; 1.9366x vs baseline; 1.9366x over previous
import jax
import jax.numpy as jnp
from jax import lax
from jax.experimental import pallas as pl
from jax.experimental.pallas import tpu as pltpu

N_DEV = 4
B, Sq, Hq, Dh = 4, 256, 8, 128
D = Hq * Dh
Skv = 1024
SCALE = 0.08838834764831843

BF = jnp.bfloat16
F32 = jnp.float32


def kernel(x, Wq, Wo, K_ext, V_ext):
    x2 = x.reshape(B * Sq, D)

    def body(x_ref, wq_ref, wo_ref, k_ref, v_ref, out_ref,
             qs, wqs, wos, o_acc, m_acc, l_acc, att_s,
             k_buf, v_buf,
             co, cm, cl, cag,
             kv_sems,
             so_sem, ro_sem, sm_sem, rm_sem, sl_sem, rl_sem,
             sag_sem, rag_sem):
        my = lax.axis_index("i")
        left = lax.rem(my + 3, N_DEV)
        right = lax.rem(my + 1, N_DEV)

        barrier = pltpu.get_barrier_semaphore()
        for nbr in (left, right):
            pl.semaphore_signal(barrier, inc=1, device_id=(nbr,),
                                device_id_type=pl.DeviceIdType.MESH)
        pl.semaphore_wait(barrier, 2)

        def chunk_b(t):
            return lax.rem(my + (N_DEV - 1 - t), N_DEV)

        def start_kv(it):
            t, h = divmod(it, Hq)
            b = chunk_b(t)
            slot = it % 2
            pltpu.make_async_copy(
                k_ref.at[b, :, h, :], k_buf.at[slot], kv_sems.at[slot, 0]
            ).start()
            pltpu.make_async_copy(
                v_ref.at[b, :, h, :], v_buf.at[slot], kv_sems.at[slot, 1]
            ).start()

        def wait_kv(slot):
            pltpu.make_async_copy(
                k_ref.at[0, :, 0, :], k_buf.at[slot], kv_sems.at[slot, 0]
            ).wait()
            pltpu.make_async_copy(
                v_ref.at[0, :, 0, :], v_buf.at[slot], kv_sems.at[slot, 1]
            ).wait()

        def ring_rdmas(s):
            ss, rs = s % 2, (s + 1) % 2
            return [
                pltpu.make_async_remote_copy(
                    src_ref=buf.at[ss], dst_ref=buf.at[rs],
                    send_sem=snd.at[ss], recv_sem=rcv.at[rs],
                    device_id=(right,), device_id_type=pl.DeviceIdType.MESH)
                for buf, snd, rcv in (
                    (co, so_sem, ro_sem),
                    (cm, sm_sem, rm_sem),
                    (cl, sl_sem, rl_sem),
                )
            ]

        start_kv(0)
        wqs[...] = wq_ref[...].astype(BF)
        wos[...] = wo_ref[...].astype(BF)
        for b in range(B):
            rows = pl.ds(b * Sq, Sq)
            qs[rows, :] = (lax.dot_general(
                x_ref[rows, :].astype(BF), wqs[...],
                (((1,), (0,)), ((), ())), preferred_element_type=F32,
            ) * SCALE).astype(BF)

        hop = None
        fin = None
        for t in range(N_DEV):
            for h in range(Hq):
                it = t * Hq + h
                slot = it % 2
                wait_kv(slot)
                if it + 1 < N_DEV * Hq:
                    start_kv(it + 1)
                b = chunk_b(t)
                q = qs[pl.ds(b * Sq, Sq), h * Dh:(h + 1) * Dh]
                k = k_buf[slot].astype(BF)
                s_ = lax.dot_general(
                    q, k, (((1,), (1,)), ((), ())),
                    preferred_element_type=F32)
                m = jnp.max(s_, axis=1, keepdims=True)
                p = jnp.exp(s_ - m)
                l = jnp.sum(p, axis=1, keepdims=True)
                v = v_buf[slot].astype(BF)
                o = lax.dot_general(
                    p.astype(BF), v, (((1,), (0,)), ((), ())),
                    preferred_element_type=F32)
                if t == 0:
                    co[0, :, h * Dh:(h + 1) * Dh] = o.astype(BF)
                    cm[0, :, h:h + 1] = m
                    cl[0, :, h:h + 1] = l
                else:
                    rows = pl.ds(t * Sq, Sq)
                    o_acc[rows, h * Dh:(h + 1) * Dh] = o
                    m_acc[rows, h:h + 1] = m
                    l_acc[rows, h:h + 1] = l

            if t == 0:
                hop = ring_rdmas(0)
                for r in hop:
                    r.start()
                continue

            s = t - 1
            rs = (s + 1) % 2
            for r in hop:
                r.wait()
            rows = pl.ds(t * Sq, Sq)
            m_loc = m_acc[rows, :]
            l_loc = l_acc[rows, :]
            m_r = cm[rs]
            l_r = cl[rs]
            m_new = jnp.maximum(m_loc, m_r)
            a_loc = jnp.exp(m_loc - m_new)
            a_r = jnp.exp(m_r - m_new)
            l_new = l_loc * a_loc + l_r * a_r
            if s < N_DEV - 2:
                cm[rs] = m_new
                cl[rs] = l_new
            for h in range(Hq):
                cols = pl.ds(h * Dh, Dh)
                o_h = (o_acc[rows, cols] * a_loc[:, h:h + 1]
                       + co[rs, :, cols].astype(F32) * a_r[:, h:h + 1])
                if s < N_DEV - 2:
                    co[rs, :, cols] = o_h.astype(BF)
                else:
                    att_s[:, cols] = (o_h / l_new[:, h:h + 1]).astype(BF)
            if s < N_DEV - 2:
                hop = ring_rdmas(s + 1)
                for r in hop:
                    r.start()
            else:
                fin = lax.dot_general(
                    att_s[...], wos[...], (((1,), (0,)), ((), ())),
                    preferred_element_type=F32)

        def ag_rdma(src_slot, dst_slot, send_i, dev):
            return pltpu.make_async_remote_copy(
                src_ref=cag.at[src_slot], dst_ref=cag.at[dst_slot],
                send_sem=sag_sem.at[send_i], recv_sem=rag_sem.at[dst_slot],
                device_id=(dev,), device_id_type=pl.DeviceIdType.MESH)

        cag[0] = fin.astype(BF)
        send_l = ag_rdma(0, 2, 0, left)
        send_r = ag_rdma(0, 1, 1, right)
        send_l.start()
        send_r.start()
        out_ref[pl.ds(my * Sq, Sq), :] = fin

        recv1 = ag_rdma(0, 1, 0, left)
        recv1.wait_recv()
        fwd = ag_rdma(1, 3, 2, right)
        fwd.start()
        out_ref[pl.ds(left * Sq, Sq), :] = cag[1].astype(F32)

        recv2 = ag_rdma(0, 2, 0, left)
        recv2.wait_recv()
        out_ref[pl.ds(right * Sq, Sq), :] = cag[2].astype(F32)

        recv3 = ag_rdma(0, 3, 0, left)
        recv3.wait_recv()
        diag = lax.rem(my + 2, N_DEV)
        out_ref[pl.ds(diag * Sq, Sq), :] = cag[3].astype(F32)

        send_l.wait_send()
        send_r.wait_send()
        fwd.wait_send()

    out = pl.pallas_call(
        body,
        out_shape=jax.ShapeDtypeStruct((B * Sq, D), F32),
        in_specs=[
            pl.BlockSpec(memory_space=pltpu.VMEM),
            pl.BlockSpec(memory_space=pltpu.VMEM),
            pl.BlockSpec(memory_space=pltpu.VMEM),
            pl.BlockSpec(memory_space=pltpu.MemorySpace.HBM),
            pl.BlockSpec(memory_space=pltpu.MemorySpace.HBM),
        ],
        out_specs=pl.BlockSpec(memory_space=pltpu.VMEM),
        scratch_shapes=[
            pltpu.VMEM((B * Sq, D), BF),
            pltpu.VMEM((D, D), BF),
            pltpu.VMEM((D, D), BF),
            pltpu.VMEM((B * Sq, D), F32),
            pltpu.VMEM((B * Sq, Hq), F32),
            pltpu.VMEM((B * Sq, Hq), F32),
            pltpu.VMEM((Sq, D), BF),
            pltpu.VMEM((2, Skv, Dh), F32),
            pltpu.VMEM((2, Skv, Dh), F32),
            pltpu.VMEM((2, Sq, D), BF),
            pltpu.VMEM((2, Sq, Hq), F32),
            pltpu.VMEM((2, Sq, Hq), F32),
            pltpu.VMEM((4, Sq, D), BF),
            pltpu.SemaphoreType.DMA((2, 2)),
            pltpu.SemaphoreType.DMA((2,)),
            pltpu.SemaphoreType.DMA((2,)),
            pltpu.SemaphoreType.DMA((2,)),
            pltpu.SemaphoreType.DMA((2,)),
            pltpu.SemaphoreType.DMA((2,)),
            pltpu.SemaphoreType.DMA((2,)),
            pltpu.SemaphoreType.DMA((3,)),
            pltpu.SemaphoreType.DMA((4,)),
        ],
        compiler_params=pltpu.CompilerParams(
            collective_id=0,
            vmem_limit_bytes=60 * 1024 * 1024,
        ),
    )(x2, Wq, Wo, K_ext, V_ext)
    return out.reshape(B, Sq, D)


# device time: 63802 ns/iter; 2.3290x vs baseline; 1.2026x over previous
import jax
import jax.numpy as jnp
from jax import lax
from jax.experimental import pallas as pl
from jax.experimental.pallas import tpu as pltpu

N_DEV = 4
B, Sq, Hq, Dh = 4, 256, 8, 128
D = Hq * Dh
Skv = 1024
SCALE = 0.08838834764831843

BF = jnp.bfloat16
F32 = jnp.float32


def kernel(x, Wq, Wo, K_ext, V_ext):
    x2 = x.reshape(B * Sq, D)

    def body(x_ref, wq_ref, wo_ref, k_ref, v_ref, out_ref,
             qs, wqs, wos, o_acc, l_acc, att_s,
             k_buf, v_buf,
             co, cl, cag,
             kv_sems,
             so_sem, ro_sem, sl_sem, rl_sem,
             sag_sem, rag_sem):
        my = lax.axis_index("i")
        left = lax.rem(my + 3, N_DEV)
        right = lax.rem(my + 1, N_DEV)

        barrier = pltpu.get_barrier_semaphore()
        for nbr in (left, right):
            pl.semaphore_signal(barrier, inc=1, device_id=(nbr,),
                                device_id_type=pl.DeviceIdType.MESH)
        pl.semaphore_wait(barrier, 2)

        def chunk_b(t):
            return lax.rem(my + (N_DEV - 1 - t), N_DEV)

        def start_kv(it):
            t, h = divmod(it, Hq)
            b = chunk_b(t)
            slot = it % 6
            pltpu.make_async_copy(
                k_ref.at[b, :, h, :], k_buf.at[slot], kv_sems.at[slot, 0]
            ).start()
            pltpu.make_async_copy(
                v_ref.at[b, :, h, :], v_buf.at[slot], kv_sems.at[slot, 1]
            ).start()

        def wait_kv(slot):
            pltpu.make_async_copy(
                k_ref.at[0, :, 0, :], k_buf.at[slot], kv_sems.at[slot, 0]
            ).wait()
            pltpu.make_async_copy(
                v_ref.at[0, :, 0, :], v_buf.at[slot], kv_sems.at[slot, 1]
            ).wait()

        def ring_rdmas(s):
            ss, rs = s % 2, (s + 1) % 2
            return [
                pltpu.make_async_remote_copy(
                    src_ref=buf.at[ss], dst_ref=buf.at[rs],
                    send_sem=snd.at[ss], recv_sem=rcv.at[rs],
                    device_id=(right,), device_id_type=pl.DeviceIdType.MESH)
                for buf, snd, rcv in (
                    (co, so_sem, ro_sem),
                    (cl, sl_sem, rl_sem),
                )
            ]

        for it0 in range(5):
            start_kv(it0)
        wqs[...] = wq_ref[...].astype(BF)
        wos[...] = wo_ref[...].astype(BF)
        for b in range(B):
            rows = pl.ds(b * Sq, Sq)
            qs[rows, :] = (lax.dot_general(
                x_ref[rows, :].astype(BF), wqs[...],
                (((1,), (0,)), ((), ())), preferred_element_type=F32,
            ) * SCALE).astype(BF)

        hop = None
        fin = None
        for t in range(N_DEV):
            b = chunk_b(t)
            for h in range(Hq):
                it = t * Hq + h
                slot = it % 6
                wait_kv(slot)
                if it + 5 < N_DEV * Hq:
                    start_kv(it + 5)
                q = qs[pl.ds(b * Sq, Sq), h * Dh:(h + 1) * Dh]
                k = k_buf[slot].astype(BF)
                s_ = lax.dot_general(
                    q, k, (((1,), (1,)), ((), ())),
                    preferred_element_type=F32)
                p = jnp.exp(s_)
                l = jnp.sum(p, axis=1, keepdims=True)
                v = v_buf[slot].astype(BF)
                o = lax.dot_general(
                    p.astype(BF), v, (((1,), (0,)), ((), ())),
                    preferred_element_type=F32)
                if t == 0:
                    co[0, :, h * Dh:(h + 1) * Dh] = o.astype(BF)
                    cl[0, :, h:h + 1] = l
                else:
                    rows = pl.ds(t * Sq, Sq)
                    o_acc[rows, h * Dh:(h + 1) * Dh] = o
                    l_acc[rows, h:h + 1] = l

            if t == 0:
                hop = ring_rdmas(0)
                for r in hop:
                    r.start()
                continue

            s = t - 1
            rs = (s + 1) % 2
            for r in hop:
                r.wait()
            rows = pl.ds(t * Sq, Sq)
            l_new = l_acc[rows, :] + cl[rs]
            if s < N_DEV - 2:
                cl[rs] = l_new
            for h in range(Hq):
                cols = pl.ds(h * Dh, Dh)
                o_h = o_acc[rows, cols] + co[rs, :, cols].astype(F32)
                if s < N_DEV - 2:
                    co[rs, :, cols] = o_h.astype(BF)
                else:
                    att_s[:, cols] = (o_h / l_new[:, h:h + 1]).astype(BF)
            if s < N_DEV - 2:
                hop = ring_rdmas(s + 1)
                for r in hop:
                    r.start()
            else:
                fin = lax.dot_general(
                    att_s[...], wos[...], (((1,), (0,)), ((), ())),
                    preferred_element_type=F32)

        diag = lax.rem(my + 2, N_DEV)

        def ag_rdma(dst_slot, send_i, dev):
            return pltpu.make_async_remote_copy(
                src_ref=cag.at[my], dst_ref=cag.at[dst_slot],
                send_sem=sag_sem.at[send_i], recv_sem=rag_sem.at[dst_slot],
                device_id=(dev,), device_id_type=pl.DeviceIdType.MESH)

        cag[my] = fin.astype(BF)
        sends = [ag_rdma(my, i, dev)
                 for i, dev in enumerate((left, right, diag))]
        for snd in sends:
            snd.start()
        out_ref[pl.ds(my * Sq, Sq), :] = fin

        for origin in (left, right, diag):
            recv = pltpu.make_async_remote_copy(
                src_ref=cag.at[my], dst_ref=cag.at[origin],
                send_sem=sag_sem.at[0], recv_sem=rag_sem.at[origin],
                device_id=(left,), device_id_type=pl.DeviceIdType.MESH)
            recv.wait_recv()
            out_ref[pl.ds(origin * Sq, Sq), :] = cag[origin].astype(F32)

        for snd in sends:
            snd.wait_send()

    out = pl.pallas_call(
        body,
        out_shape=jax.ShapeDtypeStruct((B * Sq, D), F32),
        in_specs=[
            pl.BlockSpec(memory_space=pltpu.VMEM),
            pl.BlockSpec(memory_space=pltpu.VMEM),
            pl.BlockSpec(memory_space=pltpu.VMEM),
            pl.BlockSpec(memory_space=pltpu.MemorySpace.HBM),
            pl.BlockSpec(memory_space=pltpu.MemorySpace.HBM),
        ],
        out_specs=pl.BlockSpec(memory_space=pltpu.VMEM),
        scratch_shapes=[
            pltpu.VMEM((B * Sq, D), BF),
            pltpu.VMEM((D, D), BF),
            pltpu.VMEM((D, D), BF),
            pltpu.VMEM((B * Sq, D), F32),
            pltpu.VMEM((B * Sq, Hq), F32),
            pltpu.VMEM((Sq, D), BF),
            pltpu.VMEM((6, Skv, Dh), F32),
            pltpu.VMEM((6, Skv, Dh), F32),
            pltpu.VMEM((2, Sq, D), BF),
            pltpu.VMEM((2, Sq, Hq), F32),
            pltpu.VMEM((4, Sq, D), BF),
            pltpu.SemaphoreType.DMA((6, 2)),
            pltpu.SemaphoreType.DMA((2,)),
            pltpu.SemaphoreType.DMA((2,)),
            pltpu.SemaphoreType.DMA((2,)),
            pltpu.SemaphoreType.DMA((2,)),
            pltpu.SemaphoreType.DMA((3,)),
            pltpu.SemaphoreType.DMA((4,)),
        ],
        compiler_params=pltpu.CompilerParams(
            collective_id=0,
            vmem_limit_bytes=60 * 1024 * 1024,
        ),
    )(x2, Wq, Wo, K_ext, V_ext)
    return out.reshape(B, Sq, D)
